# baseline (speedup 1.0000x reference)
import functools

import jax
import jax.numpy as jnp
from jax import lax
from jax.experimental import pallas as pl
from jax.experimental.pallas import tpu as pltpu
from jax.experimental.pallas import tpu_sc as plsc

N = 10000
E = 320000
DI = 128
DO = 128
DE = 16

NC = 2
NS = 16
NW = NC * NS
CH = 80
NPAD = 10240
NPS = NPAD // NS

SLICES = 5
ESL = E // SLICES

_MESH = plsc.VectorSubcoreMesh(core_axis_name="c", subcore_axis_name="s")


def _make_gather(e_sl):
    epw = e_sl // NW
    nchunk = epw // CH
    nturn = ((nchunk + 2 + 3) // 4) * 4

    @functools.partial(
        pl.kernel,
        out_type=(
            jax.ShapeDtypeStruct((e_sl, DI), jnp.float32),
            jax.ShapeDtypeStruct((e_sl, DI), jnp.float32),
        ),
        mesh=_MESH,
        scratch_types=(
            [pltpu.VMEM((CH,), jnp.int32)] * 8
            + [pltpu.VMEM((CH, DI), jnp.float32)] * 4
            + [pltpu.VMEM_SHARED((NPAD, DI), jnp.float32)]
            + [pltpu.SemaphoreType.DMA] * 8
        ),
    )
    def gather(x_hbm, dst_hbm, src_hbm, xi_hbm, xj_hbm,
               di0, di1, di2, di3, si0, si1, si2, si3,
               ri0, ri1, rj0, rj1, tab_sh,
               sa0, sa1, sa2, sa3, sg0, sg1, sw0, sw1):
        dib = (di0, di1, di2, di3)
        sib = (si0, si1, si2, si3)
        rib = (ri0, ri1)
        rjb = (rj0, rj1)
        sa = (sa0, sa1, sa2, sa3)
        sg = (sg0, sg1)
        sw = (sw0, sw1)

        cid = lax.axis_index("c")
        sid = lax.axis_index("s")
        wid = sid * NC + cid
        base = wid * epw

        pltpu.sync_copy(x_hbm.at[pl.ds(sid * NPS, NPS)],
                        tab_sh.at[pl.ds(sid * NPS, NPS)])
        plsc.subcore_barrier()

        def fire_idx(c, q):
            off = base + c * CH
            pltpu.async_copy(dst_hbm.at[pl.ds(off, CH)], dib[q], sa[q])
            pltpu.async_copy(src_hbm.at[pl.ds(off, CH)], sib[q], sa[q])

        fire_idx(0, 0)
        fire_idx(1, 1)

        @pl.loop(0, nturn, step=4)
        def _turns(k):
            for db in range(4):
                c = k + db
                q = db
                r = db % 2
                qn = (db + 2) % 4

                @pl.when(c < nchunk)
                def _():
                    @pl.when(c >= 2)
                    def _():
                        pltpu.make_async_copy(
                            rib[r], xi_hbm.at[pl.ds(0, CH)], sw[r]).wait()
                        pltpu.make_async_copy(
                            rjb[r], xj_hbm.at[pl.ds(0, CH)], sw[r]).wait()
                    pltpu.make_async_copy(
                        dst_hbm.at[pl.ds(0, CH)], dib[q], sa[q]).wait()
                    pltpu.make_async_copy(
                        src_hbm.at[pl.ds(0, CH)], sib[q], sa[q]).wait()
                    pltpu.async_copy(tab_sh.at[dib[q]], rib[r], sg[r])
                    pltpu.async_copy(tab_sh.at[sib[q]], rjb[r], sg[r])

                    @pl.when(c + 2 < nchunk)
                    def _():
                        fire_idx(c + 2, qn)

                @pl.when(jnp.logical_and(c >= 1, c <= nchunk))
                def _():
                    cm = c - 1
                    rm = (db + 1) % 2
                    offm = base + cm * CH
                    pltpu.make_async_copy(
                        x_hbm.at[pl.ds(0, CH)], rib[rm], sg[rm]).wait()
                    pltpu.make_async_copy(
                        x_hbm.at[pl.ds(0, CH)], rjb[rm], sg[rm]).wait()
                    pltpu.async_copy(rib[rm], xi_hbm.at[pl.ds(offm, CH)],
                                     sw[rm])
                    pltpu.async_copy(rjb[rm], xj_hbm.at[pl.ds(offm, CH)],
                                     sw[rm])

        for r in range(2):
            pltpu.make_async_copy(rib[r], xi_hbm.at[pl.ds(0, CH)],
                                  sw[r]).wait()
            pltpu.make_async_copy(rjb[r], xj_hbm.at[pl.ds(0, CH)],
                                  sw[r]).wait()

    return gather


def _make_scatter(e_sl, n_in):
    epw = e_sl // NW
    nchunk = epw // CH
    nturn = ((nchunk + 3) // 4) * 4

    @functools.partial(
        pl.kernel,
        out_type=jax.ShapeDtypeStruct((NC, NPAD, DO), jnp.float32),
        mesh=_MESH,
        scratch_types=(
            [pltpu.VMEM((CH,), jnp.int32)] * 4
            + [pltpu.VMEM((CH, DO), jnp.float32)] * 4
            + [pltpu.VMEM_SHARED((NPAD, DO), jnp.float32)]
            + [pltpu.SemaphoreType.DMA] * 6
        ),
    )
    def scatter(*args):
        ms = args[:n_in]
        dsts = args[n_in:2 * n_in]
        zeros_hbm = args[2 * n_in]
        out_hbm = args[2 * n_in + 1]
        (ix0, ix1, ix2, ix3, rw0, rw1, rw2, rw3, acc_sh,
         sl0, sl1, sl2, sl3, ss0, ss1) = args[2 * n_in + 2:]
        ixb = (ix0, ix1, ix2, ix3)
        rwb = (rw0, rw1, rw2, rw3)
        sl = (sl0, sl1, sl2, sl3)
        ss = (ss0, ss1)

        cid = lax.axis_index("c")
        sid = lax.axis_index("s")
        wid = sid * NC + cid
        base = wid * epw

        pltpu.sync_copy(zeros_hbm.at[pl.ds(sid * NPS, NPS)],
                        acc_sh.at[pl.ds(sid * NPS, NPS)])
        plsc.subcore_barrier()

        for m_hbm, dst_hbm in zip(ms, dsts):
            def fire_load(c, q, m_hbm=m_hbm, dst_hbm=dst_hbm):
                off = base + c * CH
                pltpu.async_copy(dst_hbm.at[pl.ds(off, CH)], ixb[q], sl[q])
                pltpu.async_copy(m_hbm.at[pl.ds(off, CH)], rwb[q], sl[q])

            fire_load(0, 0)
            fire_load(1, 1)

            @pl.loop(0, nturn, step=4)
            def _turns(k, m_hbm=m_hbm, dst_hbm=dst_hbm, fire_load=fire_load):
                for db in range(4):
                    c = k + db
                    q = db
                    s = db % 2
                    qn = (db + 2) % 4

                    @pl.when(c < nchunk)
                    def _():
                        @pl.when(c >= 2)
                        def _():
                            pltpu.make_async_copy(
                                m_hbm.at[pl.ds(0, CH)], rwb[qn], ss[s]).wait()
                        pltpu.make_async_copy(
                            dst_hbm.at[pl.ds(0, CH)], ixb[q], sl[q]).wait()
                        pltpu.make_async_copy(
                            m_hbm.at[pl.ds(0, CH)], rwb[q], sl[q]).wait()
                        pltpu.async_copy(rwb[q], acc_sh.at[ixb[q]], ss[s],
                                         add=True)

                        @pl.when(c + 2 < nchunk)
                        def _():
                            fire_load(c + 2, qn)

            for s in range(2):
                pltpu.make_async_copy(m_hbm.at[pl.ds(0, CH)], rwb[s],
                                      ss[s]).wait()

        plsc.subcore_barrier()
        pltpu.sync_copy(acc_sh.at[pl.ds(sid * NPS, NPS)],
                        out_hbm.at[cid, pl.ds(sid * NPS, NPS)])

    return scatter


_gather_sl = _make_gather(ESL)
_scatter_a = _make_scatter(ESL, 3)
_scatter_b = _make_scatter(ESL, 2)


def _mish(v):
    z = 1.0 + jnp.exp(jnp.minimum(v, 15.0))
    z2 = z * z
    return v * (z2 - 1.0) / (z2 + 1.0)


def _msg_body(xi_ref, xj_ref, ea_ref, w1a_ref, w1b_ref, b1_ref, w2_ref,
              b2_ref, wea_ref, web_ref, be1_ref, m_ref):
    xi = xi_ref[...].astype(jnp.bfloat16)
    xj = xj_ref[...].astype(jnp.bfloat16)
    hp = (jnp.dot(xi, w1a_ref[...], preferred_element_type=jnp.float32)
          + jnp.dot(xj, w1b_ref[...], preferred_element_type=jnp.float32)
          + b1_ref[...])
    h = jnp.dot(_mish(hp).astype(jnp.bfloat16), w2_ref[...],
                preferred_element_type=jnp.float32) + b2_ref[...]
    gp = (jnp.dot(ea_ref[...].astype(jnp.bfloat16), wea_ref[...],
                  preferred_element_type=jnp.float32)
          + jnp.dot(xj, web_ref[...], preferred_element_type=jnp.float32)
          + be1_ref[...])
    m_ref[...] = h * _mish(gp)


_BLK = 2000


def _tc_messages(xi, xj, ea, w1a, w1b, b1, w2, b2, wea, web, be1):
    e_sl = xi.shape[0]
    grid = (e_sl // _BLK,)
    full = lambda shape: pl.BlockSpec(shape, lambda i: (0, 0))
    return pl.pallas_call(
        _msg_body,
        grid=grid,
        in_specs=[
            pl.BlockSpec((_BLK, DI), lambda i: (i, 0)),
            pl.BlockSpec((_BLK, DI), lambda i: (i, 0)),
            pl.BlockSpec((_BLK, DE), lambda i: (i, 0)),
            full((DI, DO)), full((DI, DO)), full((1, DO)),
            full((DO, DO)), full((1, DO)),
            full((DE, DO)), full((DI, DO)), full((1, DO)),
        ],
        out_specs=pl.BlockSpec((_BLK, DO), lambda i: (i, 0)),
        out_shape=jax.ShapeDtypeStruct((e_sl, DO), jnp.float32),
    )(xi, xj, ea, w1a, w1b, b1, w2, b2, wea, web, be1)


def _fin_body(pa_ref, pb_ref, x_ref, wr_ref, br_ref, g_ref, b_ref, out_ref):
    aggr = (pa_ref[0, :N, :] + pa_ref[1, :N, :]
            + pb_ref[0, :N, :] + pb_ref[1, :N, :])
    out = aggr + jnp.dot(x_ref[...], wr_ref[...],
                         preferred_element_type=jnp.float32) + br_ref[...]
    mu = jnp.mean(out, axis=0, keepdims=True)
    var = jnp.mean((out - mu) ** 2, axis=0, keepdims=True)
    out_ref[...] = (out - mu) * jax.lax.rsqrt(var + 1e-5) * g_ref[...] + b_ref[...]


def _tc_finish(pa, pb, x, wr, br, gamma, beta):
    return pl.pallas_call(
        _fin_body,
        out_shape=jax.ShapeDtypeStruct((N, DO), jnp.float32),
    )(pa, pb, x, wr, br, gamma, beta)


def kernel(x, edge_index, edge_attr, W1, b1, W2, b2, We1, be1, Wr, br,
           gamma, beta):
    src = edge_index[0].astype(jnp.int32)
    dst = edge_index[1].astype(jnp.int32)

    xpad = jnp.zeros((NPAD, DI), jnp.float32).at[:N].set(x)

    w1a = W1[:DI].astype(jnp.bfloat16)
    w1b = W1[DI:].astype(jnp.bfloat16)
    w2 = W2.astype(jnp.bfloat16)
    wea = We1[:DE].astype(jnp.bfloat16)
    web = We1[DE:].astype(jnp.bfloat16)
    b1r = b1.reshape(1, DO)
    b2r = b2.reshape(1, DO)
    be1r = be1.reshape(1, DO)

    ms, dsts = [], []
    for s in range(SLICES):
        lo = s * ESL
        dst_s = lax.slice_in_dim(dst, lo, lo + ESL)
        src_s = lax.slice_in_dim(src, lo, lo + ESL)
        ea_s = lax.slice_in_dim(edge_attr, lo, lo + ESL)
        xi_s, xj_s = _gather_sl(xpad, dst_s, src_s)
        ms.append(_tc_messages(xi_s, xj_s, ea_s,
                               w1a, w1b, b1r, w2, b2r, wea, web, be1r))
        dsts.append(dst_s)

    zeros = jnp.zeros((NPAD, DO), jnp.float32)
    pa = _scatter_a(ms[0], ms[1], ms[2], dsts[0], dsts[1], dsts[2], zeros)
    pb = _scatter_b(ms[3], ms[4], dsts[3], dsts[4], zeros)

    out = _tc_finish(pa, pb, x, Wr, br.reshape(1, DO),
                     gamma.reshape(1, DO), beta.reshape(1, DO))
    return (out, edge_index, edge_attr)


# speedup vs baseline: 1.0561x; 1.0561x over previous
import functools

import jax
import jax.numpy as jnp
from jax import lax
from jax.experimental import pallas as pl
from jax.experimental.pallas import tpu as pltpu
from jax.experimental.pallas import tpu_sc as plsc

N = 10000
E = 320000
DI = 128
DO = 128
DE = 16

NC = 2
NS = 16
NW = NC * NS
CH = 80
NPAD = 10240
DP = DI // 2
NPS = NPAD // NS

SLICES = 5
ESL = E // SLICES

_MESH = plsc.VectorSubcoreMesh(core_axis_name="c", subcore_axis_name="s")


def _make_gather(e_sl):
    epw = e_sl // NW
    nchunk = epw // CH
    nturn = ((nchunk + 2 + 3) // 4) * 4

    @functools.partial(
        pl.kernel,
        out_type=(
            jax.ShapeDtypeStruct((e_sl, DI), jnp.float32),
            jax.ShapeDtypeStruct((e_sl, DI), jnp.float32),
        ),
        mesh=_MESH,
        scratch_types=(
            [pltpu.VMEM((CH,), jnp.int32)] * 8
            + [pltpu.VMEM((CH, DI), jnp.float32)] * 4
            + [pltpu.VMEM_SHARED((NPAD, DI), jnp.float32)]
            + [pltpu.SemaphoreType.DMA] * 8
        ),
    )
    def gather(x_hbm, dst_hbm, src_hbm, xi_hbm, xj_hbm,
               di0, di1, di2, di3, si0, si1, si2, si3,
               ri0, ri1, rj0, rj1, tab_sh,
               sa0, sa1, sa2, sa3, sg0, sg1, sw0, sw1):
        dib = (di0, di1, di2, di3)
        sib = (si0, si1, si2, si3)
        rib = (ri0, ri1)
        rjb = (rj0, rj1)
        sa = (sa0, sa1, sa2, sa3)
        sg = (sg0, sg1)
        sw = (sw0, sw1)

        cid = lax.axis_index("c")
        sid = lax.axis_index("s")
        wid = sid * NC + cid
        base = wid * epw

        pltpu.sync_copy(x_hbm.at[pl.ds(sid * NPS, NPS)],
                        tab_sh.at[pl.ds(sid * NPS, NPS)])
        plsc.subcore_barrier()

        def fire_idx(c, q):
            off = base + c * CH
            pltpu.async_copy(dst_hbm.at[pl.ds(off, CH)], dib[q], sa[q])
            pltpu.async_copy(src_hbm.at[pl.ds(off, CH)], sib[q], sa[q])

        fire_idx(0, 0)
        fire_idx(1, 1)

        @pl.loop(0, nturn, step=4)
        def _turns(k):
            for db in range(4):
                c = k + db
                q = db
                r = db % 2
                qn = (db + 2) % 4

                @pl.when(c < nchunk)
                def _():
                    @pl.when(c >= 2)
                    def _():
                        pltpu.make_async_copy(
                            rib[r], xi_hbm.at[pl.ds(0, CH)], sw[r]).wait()
                        pltpu.make_async_copy(
                            rjb[r], xj_hbm.at[pl.ds(0, CH)], sw[r]).wait()
                    pltpu.make_async_copy(
                        dst_hbm.at[pl.ds(0, CH)], dib[q], sa[q]).wait()
                    pltpu.make_async_copy(
                        src_hbm.at[pl.ds(0, CH)], sib[q], sa[q]).wait()
                    pltpu.async_copy(tab_sh.at[dib[q]], rib[r], sg[r])
                    pltpu.async_copy(tab_sh.at[sib[q]], rjb[r], sg[r])

                    @pl.when(c + 2 < nchunk)
                    def _():
                        fire_idx(c + 2, qn)

                @pl.when(jnp.logical_and(c >= 1, c <= nchunk))
                def _():
                    cm = c - 1
                    rm = (db + 1) % 2
                    offm = base + cm * CH
                    pltpu.make_async_copy(
                        x_hbm.at[pl.ds(0, CH)], rib[rm], sg[rm]).wait()
                    pltpu.make_async_copy(
                        x_hbm.at[pl.ds(0, CH)], rjb[rm], sg[rm]).wait()
                    pltpu.async_copy(rib[rm], xi_hbm.at[pl.ds(offm, CH)],
                                     sw[rm])
                    pltpu.async_copy(rjb[rm], xj_hbm.at[pl.ds(offm, CH)],
                                     sw[rm])

        for r in range(2):
            pltpu.make_async_copy(rib[r], xi_hbm.at[pl.ds(0, CH)],
                                  sw[r]).wait()
            pltpu.make_async_copy(rjb[r], xj_hbm.at[pl.ds(0, CH)],
                                  sw[r]).wait()

    return gather


def _make_scatter(e_sl, n_in):
    epw = e_sl // NW
    nchunk = epw // CH
    nturn = ((nchunk + 3) // 4) * 4

    @functools.partial(
        pl.kernel,
        out_type=jax.ShapeDtypeStruct((NC, NPAD, DO), jnp.float32),
        mesh=_MESH,
        scratch_types=(
            [pltpu.VMEM((CH,), jnp.int32)] * 4
            + [pltpu.VMEM((CH, DO), jnp.float32)] * 4
            + [pltpu.VMEM_SHARED((NPAD, DO), jnp.float32)]
            + [pltpu.SemaphoreType.DMA] * 6
        ),
    )
    def scatter(*args):
        ms = args[:n_in]
        dsts = args[n_in:2 * n_in]
        zeros_hbm = args[2 * n_in]
        out_hbm = args[2 * n_in + 1]
        (ix0, ix1, ix2, ix3, rw0, rw1, rw2, rw3, acc_sh,
         sl0, sl1, sl2, sl3, ss0, ss1) = args[2 * n_in + 2:]
        ixb = (ix0, ix1, ix2, ix3)
        rwb = (rw0, rw1, rw2, rw3)
        sl = (sl0, sl1, sl2, sl3)
        ss = (ss0, ss1)

        cid = lax.axis_index("c")
        sid = lax.axis_index("s")
        wid = sid * NC + cid
        base = wid * epw

        pltpu.sync_copy(zeros_hbm.at[pl.ds(sid * NPS, NPS)],
                        acc_sh.at[pl.ds(sid * NPS, NPS)])
        plsc.subcore_barrier()

        for m_hbm, dst_hbm in zip(ms, dsts):
            def fire_load(c, q, m_hbm=m_hbm, dst_hbm=dst_hbm):
                off = base + c * CH
                pltpu.async_copy(dst_hbm.at[pl.ds(off, CH)], ixb[q], sl[q])
                pltpu.async_copy(m_hbm.at[pl.ds(off, CH)], rwb[q], sl[q])

            fire_load(0, 0)
            fire_load(1, 1)

            @pl.loop(0, nturn, step=4)
            def _turns(k, m_hbm=m_hbm, dst_hbm=dst_hbm, fire_load=fire_load):
                for db in range(4):
                    c = k + db
                    q = db
                    s = db % 2
                    qn = (db + 2) % 4

                    @pl.when(c < nchunk)
                    def _():
                        @pl.when(c >= 2)
                        def _():
                            pltpu.make_async_copy(
                                m_hbm.at[pl.ds(0, CH)], rwb[qn], ss[s]).wait()
                        pltpu.make_async_copy(
                            dst_hbm.at[pl.ds(0, CH)], ixb[q], sl[q]).wait()
                        pltpu.make_async_copy(
                            m_hbm.at[pl.ds(0, CH)], rwb[q], sl[q]).wait()
                        pltpu.async_copy(rwb[q], acc_sh.at[ixb[q]], ss[s],
                                         add=True)

                        @pl.when(c + 2 < nchunk)
                        def _():
                            fire_load(c + 2, qn)

            for s in range(2):
                pltpu.make_async_copy(m_hbm.at[pl.ds(0, CH)], rwb[s],
                                      ss[s]).wait()

        plsc.subcore_barrier()
        pltpu.sync_copy(acc_sh.at[pl.ds(sid * NPS, NPS)],
                        out_hbm.at[cid, pl.ds(sid * NPS, NPS)])

    return scatter


_gather_sl = _make_gather(ESL)
_scatter_a = _make_scatter(ESL, 4)
_scatter_b = _make_scatter(ESL, 1)


def _mish(v):
    z = 1.0 + jnp.exp(jnp.minimum(v, 15.0))
    z2 = z * z
    return v * (z2 - 1.0) / (z2 + 1.0)


def _msg_body(xi_ref, xj_ref, ea_ref, w1a_ref, w1b_ref, b1_ref, w2_ref,
              b2_ref, wea_ref, web_ref, be1_ref, m_ref):
    xi = xi_ref[...].astype(jnp.bfloat16)
    xj = xj_ref[...].astype(jnp.bfloat16)
    dot = functools.partial(jnp.dot, preferred_element_type=jnp.float32)
    hp = dot(xi, w1a_ref[...]) + dot(xj, w1b_ref[...]) + b1_ref[...]
    h = dot(_mish(hp).astype(jnp.bfloat16), w2_ref[...]) + b2_ref[...]
    gp = (dot(ea_ref[...].astype(jnp.bfloat16), wea_ref[...])
          + dot(xj, web_ref[...]) + be1_ref[...])
    m_ref[...] = h * _mish(gp)


_BLK = 4000


def _tc_messages(xi, xj, ea, w1a, w1b, b1, w2, b2, wea, web, be1):
    e_sl = xi.shape[0]
    grid = (e_sl // _BLK,)
    def full(shape):
        return pl.BlockSpec(shape, lambda i: (0,) * len(shape))
    return pl.pallas_call(
        _msg_body,
        grid=grid,
        in_specs=[
            pl.BlockSpec((_BLK, DI), lambda i: (i, 0)),
            pl.BlockSpec((_BLK, DI), lambda i: (i, 0)),
            pl.BlockSpec((_BLK, DE), lambda i: (i, 0)),
            full((DI, DO)), full((DI, DO)), full((1, DO)),
            full((DO, DO)), full((1, DO)),
            full((DE, DO)), full((DI, DO)), full((1, DO)),
        ],
        out_specs=pl.BlockSpec((_BLK, DO), lambda i: (i, 0)),
        out_shape=jax.ShapeDtypeStruct((e_sl, DO), jnp.float32),
    )(xi, xj, ea, w1a, w1b, b1, w2, b2, wea, web, be1)


def _fin_body(pa_ref, pb_ref, x_ref, wr_ref, br_ref, g_ref, b_ref, out_ref):
    aggr = (pa_ref[0, :N, :] + pa_ref[1, :N, :]
            + pb_ref[0, :N, :] + pb_ref[1, :N, :])
    out = aggr + jnp.dot(x_ref[...], wr_ref[...],
                         preferred_element_type=jnp.float32) + br_ref[...]
    mu = jnp.mean(out, axis=0, keepdims=True)
    var = jnp.mean((out - mu) ** 2, axis=0, keepdims=True)
    out_ref[...] = (out - mu) * jax.lax.rsqrt(var + 1e-5) * g_ref[...] + b_ref[...]


def _tc_finish(pa, pb, x, wr, br, gamma, beta):
    return pl.pallas_call(
        _fin_body,
        out_shape=jax.ShapeDtypeStruct((N, DO), jnp.float32),
    )(pa, pb, x, wr, br, gamma, beta)


def kernel(x, edge_index, edge_attr, W1, b1, W2, b2, We1, be1, Wr, br,
           gamma, beta):
    src = edge_index[0].astype(jnp.int32)
    dst = edge_index[1].astype(jnp.int32)

    xpad = jnp.zeros((NPAD, DI), jnp.float32).at[:N].set(x)

    w1a = W1[:DI].astype(jnp.bfloat16)
    w1b = W1[DI:].astype(jnp.bfloat16)
    w2 = W2.astype(jnp.bfloat16)
    wea = We1[:DE].astype(jnp.bfloat16)
    web = We1[DE:].astype(jnp.bfloat16)
    b1r = b1.reshape(1, DO)
    b2r = b2.reshape(1, DO)
    be1r = be1.reshape(1, DO)

    ms, dsts = [], []
    for s in range(SLICES):
        lo = s * ESL
        dst_s = lax.slice_in_dim(dst, lo, lo + ESL)
        src_s = lax.slice_in_dim(src, lo, lo + ESL)
        ea_s = lax.slice_in_dim(edge_attr, lo, lo + ESL)
        xi_s, xj_s = _gather_sl(xpad, dst_s, src_s)
        ms.append(_tc_messages(xi_s, xj_s, ea_s,
                               w1a, w1b, b1r, w2, b2r, wea, web, be1r))
        dsts.append(dst_s)

    zeros = jnp.zeros((NPAD, DO), jnp.float32)
    pa = _scatter_a(ms[0], ms[1], ms[2], ms[3],
                    dsts[0], dsts[1], dsts[2], dsts[3], zeros)
    pb = _scatter_b(ms[4], dsts[4], zeros)

    out = _tc_finish(pa, pb, x, Wr, br.reshape(1, DO),
                     gamma.reshape(1, DO), beta.reshape(1, DO))
    return (out, edge_index, edge_attr)


# speedup vs baseline: 1.2665x; 1.1993x over previous
import functools

import jax
import jax.numpy as jnp
from jax import lax
from jax.experimental import pallas as pl
from jax.experimental.pallas import tpu as pltpu
from jax.experimental.pallas import tpu_sc as plsc

N = 10000
E = 320000
DI = 128
DO = 128
DE = 16

NC = 2
NS = 16
NW = NC * NS
CH = 80
NPAD = 10240
DP = DI // 2
NPS = NPAD // NS

SLICES = 5
ESL = E // SLICES

_MESH = plsc.VectorSubcoreMesh(core_axis_name="c", subcore_axis_name="s")


def _make_gather(e_sl, base0):
    epw = e_sl // NW
    nchunk = epw // CH
    nturn = ((nchunk + 2 + 3) // 4) * 4

    @functools.partial(
        pl.kernel,
        out_type=(
            jax.ShapeDtypeStruct((e_sl, DI), jnp.float32),
            jax.ShapeDtypeStruct((e_sl, DI), jnp.float32),
        ),
        mesh=_MESH,
        scratch_types=(
            [pltpu.VMEM((CH,), jnp.int32)] * 8
            + [pltpu.VMEM((CH, DI), jnp.float32)] * 4
            + [pltpu.VMEM_SHARED((NPAD, DI), jnp.float32)]
            + [pltpu.SemaphoreType.DMA] * 8
        ),
    )
    def gather(x_hbm, src_hbm, dst_hbm, xi_hbm, xj_hbm,
               di0, di1, di2, di3, si0, si1, si2, si3,
               ri0, ri1, rj0, rj1, tab_sh,
               sa0, sa1, sa2, sa3, sg0, sg1, sw0, sw1):
        dib = (di0, di1, di2, di3)
        sib = (si0, si1, si2, si3)
        rib = (ri0, ri1)
        rjb = (rj0, rj1)
        sa = (sa0, sa1, sa2, sa3)
        sg = (sg0, sg1)
        sw = (sw0, sw1)

        cid = lax.axis_index("c")
        sid = lax.axis_index("s")
        wid = sid * NC + cid
        base = base0 + wid * epw

        pltpu.sync_copy(x_hbm.at[pl.ds(sid * NPS, NPS)],
                        tab_sh.at[pl.ds(sid * NPS, NPS)])
        plsc.subcore_barrier()

        def fire_idx(c, q):
            off = base + c * CH
            pltpu.async_copy(dst_hbm.at[pl.ds(off, CH)], dib[q], sa[q])
            pltpu.async_copy(src_hbm.at[pl.ds(off, CH)], sib[q], sa[q])

        fire_idx(0, 0)
        fire_idx(1, 1)

        @pl.loop(0, nturn, step=4)
        def _turns(k):
            for db in range(4):
                c = k + db
                q = db
                r = db % 2
                qn = (db + 2) % 4

                @pl.when(c < nchunk)
                def _():
                    @pl.when(c >= 2)
                    def _():
                        pltpu.make_async_copy(
                            rib[r], xi_hbm.at[pl.ds(0, CH)], sw[r]).wait()
                        pltpu.make_async_copy(
                            rjb[r], xj_hbm.at[pl.ds(0, CH)], sw[r]).wait()
                    pltpu.make_async_copy(
                        dst_hbm.at[pl.ds(0, CH)], dib[q], sa[q]).wait()
                    pltpu.make_async_copy(
                        src_hbm.at[pl.ds(0, CH)], sib[q], sa[q]).wait()
                    pltpu.async_copy(tab_sh.at[dib[q]], rib[r], sg[r])
                    pltpu.async_copy(tab_sh.at[sib[q]], rjb[r], sg[r])

                    @pl.when(c + 2 < nchunk)
                    def _():
                        fire_idx(c + 2, qn)

                @pl.when(jnp.logical_and(c >= 1, c <= nchunk))
                def _():
                    cm = c - 1
                    rm = (db + 1) % 2
                    offm = base - base0 + cm * CH
                    pltpu.make_async_copy(
                        x_hbm.at[pl.ds(0, CH)], rib[rm], sg[rm]).wait()
                    pltpu.make_async_copy(
                        x_hbm.at[pl.ds(0, CH)], rjb[rm], sg[rm]).wait()
                    pltpu.async_copy(rib[rm], xi_hbm.at[pl.ds(offm, CH)],
                                     sw[rm])
                    pltpu.async_copy(rjb[rm], xj_hbm.at[pl.ds(offm, CH)],
                                     sw[rm])

        for r in range(2):
            pltpu.make_async_copy(rib[r], xi_hbm.at[pl.ds(0, CH)],
                                  sw[r]).wait()
            pltpu.make_async_copy(rjb[r], xj_hbm.at[pl.ds(0, CH)],
                                  sw[r]).wait()

    return gather


def _make_scatter(e_sl, bases):
    n_in = len(bases)
    epw = e_sl // NW
    nchunk = epw // CH
    nturn = ((nchunk + 3) // 4) * 4

    @functools.partial(
        pl.kernel,
        out_type=jax.ShapeDtypeStruct((NC, NPAD, DO), jnp.float32),
        mesh=_MESH,
        scratch_types=(
            [pltpu.VMEM((CH,), jnp.int32)] * 4
            + [pltpu.VMEM((CH, DO), jnp.float32)] * 4
            + [pltpu.VMEM_SHARED((NPAD, DO), jnp.float32)]
            + [pltpu.SemaphoreType.DMA] * 6
        ),
    )
    def scatter(*args):
        ms = args[:n_in]
        dst_hbm = args[n_in]
        zeros_hbm = args[n_in + 1]
        out_hbm = args[n_in + 2]
        (ix0, ix1, ix2, ix3, rw0, rw1, rw2, rw3, acc_sh,
         sl0, sl1, sl2, sl3, ss0, ss1) = args[n_in + 3:]
        ixb = (ix0, ix1, ix2, ix3)
        rwb = (rw0, rw1, rw2, rw3)
        sl = (sl0, sl1, sl2, sl3)
        ss = (ss0, ss1)

        cid = lax.axis_index("c")
        sid = lax.axis_index("s")
        wid = sid * NC + cid
        base = wid * epw

        pltpu.sync_copy(zeros_hbm.at[pl.ds(sid * NPS, NPS)],
                        acc_sh.at[pl.ds(sid * NPS, NPS)])
        plsc.subcore_barrier()

        for m_hbm, base0 in zip(ms, bases):
            def fire_load(c, q, m_hbm=m_hbm, base0=base0):
                off = base + c * CH
                pltpu.async_copy(dst_hbm.at[pl.ds(base0 + off, CH)],
                                 ixb[q], sl[q])
                pltpu.async_copy(m_hbm.at[pl.ds(off, CH)], rwb[q], sl[q])

            fire_load(0, 0)
            fire_load(1, 1)

            @pl.loop(0, nturn, step=4)
            def _turns(k, m_hbm=m_hbm, fire_load=fire_load):
                for db in range(4):
                    c = k + db
                    q = db
                    s = db % 2
                    qn = (db + 2) % 4

                    @pl.when(c < nchunk)
                    def _():
                        @pl.when(c >= 2)
                        def _():
                            pltpu.make_async_copy(
                                m_hbm.at[pl.ds(0, CH)], rwb[qn], ss[s]).wait()
                        pltpu.make_async_copy(
                            dst_hbm.at[pl.ds(0, CH)], ixb[q], sl[q]).wait()
                        pltpu.make_async_copy(
                            m_hbm.at[pl.ds(0, CH)], rwb[q], sl[q]).wait()
                        pltpu.async_copy(rwb[q], acc_sh.at[ixb[q]], ss[s],
                                         add=True)

                        @pl.when(c + 2 < nchunk)
                        def _():
                            fire_load(c + 2, qn)

            for s in range(2):
                pltpu.make_async_copy(m_hbm.at[pl.ds(0, CH)], rwb[s],
                                      ss[s]).wait()

        plsc.subcore_barrier()
        pltpu.sync_copy(acc_sh.at[pl.ds(sid * NPS, NPS)],
                        out_hbm.at[cid, pl.ds(sid * NPS, NPS)])

    return scatter


_gathers = [_make_gather(ESL, s * ESL) for s in range(SLICES)]
_scatter_a = _make_scatter(ESL, (0, ESL, 2 * ESL, 3 * ESL))
_scatter_b = _make_scatter(ESL, (4 * ESL,))


def _mish(v):
    z = 1.0 + jnp.exp(jnp.minimum(v, 15.0))
    z2 = z * z
    return v * (z2 - 1.0) / (z2 + 1.0)


def _msg_body(xi_ref, xj_ref, ea_ref, w1a_ref, w1b_ref, b1_ref, w2_ref,
              b2_ref, wea_ref, web_ref, be1_ref, m_ref):
    xi = xi_ref[...].astype(jnp.bfloat16)
    xj = xj_ref[...].astype(jnp.bfloat16)
    dot = functools.partial(jnp.dot, preferred_element_type=jnp.float32)
    hp = dot(xi, w1a_ref[...]) + dot(xj, w1b_ref[...]) + b1_ref[...]
    h = dot(_mish(hp).astype(jnp.bfloat16), w2_ref[...]) + b2_ref[...]
    gp = (lax.dot_general(ea_ref[...].astype(jnp.bfloat16), wea_ref[...],
                          (((0,), (0,)), ((), ())),
                          preferred_element_type=jnp.float32)
          + dot(xj, web_ref[...]) + be1_ref[...])
    m_ref[...] = h * _mish(gp)


_BLK = 3200


def _tc_messages(s, xi, xj, ea_t, w1a, w1b, b1, w2, b2, wea, web, be1):
    e_sl = xi.shape[0]
    grid = (e_sl // _BLK,)
    blk0 = s * (e_sl // _BLK)
    def full(shape):
        return pl.BlockSpec(shape, lambda i: (0,) * len(shape))
    return pl.pallas_call(
        _msg_body,
        grid=grid,
        in_specs=[
            pl.BlockSpec((_BLK, DI), lambda i: (i, 0)),
            pl.BlockSpec((_BLK, DI), lambda i: (i, 0)),
            pl.BlockSpec((DE, _BLK), lambda i: (0, blk0 + i)),
            full((DI, DO)), full((DI, DO)), full((1, DO)),
            full((DO, DO)), full((1, DO)),
            full((DE, DO)), full((DI, DO)), full((1, DO)),
        ],
        out_specs=pl.BlockSpec((_BLK, DO), lambda i: (i, 0)),
        out_shape=jax.ShapeDtypeStruct((e_sl, DO), jnp.float32),
    )(xi, xj, ea_t, w1a, w1b, b1, w2, b2, wea, web, be1)


def _fin_body(pa_ref, pb_ref, x_ref, wr_ref, br_ref, g_ref, b_ref, out_ref):
    aggr = (pa_ref[0, :N, :] + pa_ref[1, :N, :]
            + pb_ref[0, :N, :] + pb_ref[1, :N, :])
    out = aggr + jnp.dot(x_ref[...], wr_ref[...],
                         preferred_element_type=jnp.float32) + br_ref[...]
    mu = jnp.mean(out, axis=0, keepdims=True)
    var = jnp.mean((out - mu) ** 2, axis=0, keepdims=True)
    out_ref[...] = (out - mu) * jax.lax.rsqrt(var + 1e-5) * g_ref[...] + b_ref[...]


def _tc_finish(pa, pb, x, wr, br, gamma, beta):
    return pl.pallas_call(
        _fin_body,
        out_shape=jax.ShapeDtypeStruct((N, DO), jnp.float32),
    )(pa, pb, x, wr, br, gamma, beta)


def kernel(x, edge_index, edge_attr, W1, b1, W2, b2, We1, be1, Wr, br,
           gamma, beta):
    srca = edge_index[0].astype(jnp.int32)
    dsta = edge_index[1].astype(jnp.int32)
    ea_t = edge_attr.T

    xpad = jnp.zeros((NPAD, DI), jnp.float32).at[:N].set(x)

    w1a = W1[:DI].astype(jnp.bfloat16)
    w1b = W1[DI:].astype(jnp.bfloat16)
    w2 = W2.astype(jnp.bfloat16)
    wea = We1[:DE].astype(jnp.bfloat16)
    web = We1[DE:].astype(jnp.bfloat16)
    b1r = b1.reshape(1, DO)
    b2r = b2.reshape(1, DO)
    be1r = be1.reshape(1, DO)

    ms = []
    for s in range(SLICES):
        xi_s, xj_s = _gathers[s](xpad, srca, dsta)
        ms.append(_tc_messages(s, xi_s, xj_s, ea_t,
                               w1a, w1b, b1r, w2, b2r, wea, web, be1r))

    zeros = jnp.zeros((NPAD, DO), jnp.float32)
    pa = _scatter_a(ms[0], ms[1], ms[2], ms[3], dsta, zeros)
    pb = _scatter_b(ms[4], dsta, zeros)

    out = _tc_finish(pa, pb, x, Wr, br.reshape(1, DO),
                     gamma.reshape(1, DO), beta.reshape(1, DO))
    return (out, edge_index, edge_attr)


# speedup vs baseline: 1.3161x; 1.0391x over previous
import functools

import jax
import jax.numpy as jnp
from jax import lax
from jax.experimental import pallas as pl
from jax.experimental.pallas import tpu as pltpu
from jax.experimental.pallas import tpu_sc as plsc

N = 10000
E = 320000
DI = 128
DO = 128
DE = 16

NC = 2
NS = 16
NW = NC * NS
CH = 80
NPAD = 10240
DP = DI // 2
NPS = NPAD // NS

SLICES = 5
ESL = E // SLICES

_MESH = plsc.VectorSubcoreMesh(core_axis_name="c", subcore_axis_name="s")


def _make_gather(e_sl, base0):
    epw = e_sl // NW
    nchunk = epw // CH
    nturn = ((nchunk + 2 + 3) // 4) * 4

    @functools.partial(
        pl.kernel,
        out_type=(
            jax.ShapeDtypeStruct((e_sl, DI), jnp.float32),
            jax.ShapeDtypeStruct((e_sl, DI), jnp.float32),
        ),
        mesh=_MESH,
        scratch_types=(
            [pltpu.VMEM((CH,), jnp.int32)] * 8
            + [pltpu.VMEM((CH, DI), jnp.float32)] * 4
            + [pltpu.VMEM_SHARED((NPAD, DI), jnp.float32)]
            + [pltpu.SemaphoreType.DMA] * 8
        ),
    )
    def gather(x_hbm, src_hbm, dst_hbm, xi_hbm, xj_hbm,
               di0, di1, di2, di3, si0, si1, si2, si3,
               ri0, ri1, rj0, rj1, tab_sh,
               sa0, sa1, sa2, sa3, sg0, sg1, sw0, sw1):
        dib = (di0, di1, di2, di3)
        sib = (si0, si1, si2, si3)
        rib = (ri0, ri1)
        rjb = (rj0, rj1)
        sa = (sa0, sa1, sa2, sa3)
        sg = (sg0, sg1)
        sw = (sw0, sw1)

        cid = lax.axis_index("c")
        sid = lax.axis_index("s")
        wid = sid * NC + cid
        base = base0 + wid * epw

        pltpu.sync_copy(x_hbm.at[pl.ds(sid * NPS, NPS)],
                        tab_sh.at[pl.ds(sid * NPS, NPS)])
        plsc.subcore_barrier()

        def fire_idx(c, q):
            off = base + c * CH
            pltpu.async_copy(dst_hbm.at[pl.ds(off, CH)], dib[q], sa[q])
            pltpu.async_copy(src_hbm.at[pl.ds(off, CH)], sib[q], sa[q])

        fire_idx(0, 0)
        fire_idx(1, 1)

        @pl.loop(0, nturn, step=4)
        def _turns(k):
            for db in range(4):
                c = k + db
                q = db
                r = db % 2
                qn = (db + 2) % 4

                @pl.when(c < nchunk)
                def _():
                    @pl.when(c >= 2)
                    def _():
                        pltpu.make_async_copy(
                            rib[r], xi_hbm.at[pl.ds(0, CH)], sw[r]).wait()
                        pltpu.make_async_copy(
                            rjb[r], xj_hbm.at[pl.ds(0, CH)], sw[r]).wait()
                    pltpu.make_async_copy(
                        dst_hbm.at[pl.ds(0, CH)], dib[q], sa[q]).wait()
                    pltpu.make_async_copy(
                        src_hbm.at[pl.ds(0, CH)], sib[q], sa[q]).wait()
                    pltpu.async_copy(tab_sh.at[dib[q]], rib[r], sg[r])
                    pltpu.async_copy(tab_sh.at[sib[q]], rjb[r], sg[r])

                    @pl.when(c + 2 < nchunk)
                    def _():
                        fire_idx(c + 2, qn)

                @pl.when(jnp.logical_and(c >= 1, c <= nchunk))
                def _():
                    cm = c - 1
                    rm = (db + 1) % 2
                    offm = base - base0 + cm * CH
                    pltpu.make_async_copy(
                        x_hbm.at[pl.ds(0, CH)], rib[rm], sg[rm]).wait()
                    pltpu.make_async_copy(
                        x_hbm.at[pl.ds(0, CH)], rjb[rm], sg[rm]).wait()
                    pltpu.async_copy(rib[rm], xi_hbm.at[pl.ds(offm, CH)],
                                     sw[rm])
                    pltpu.async_copy(rjb[rm], xj_hbm.at[pl.ds(offm, CH)],
                                     sw[rm])

        for r in range(2):
            pltpu.make_async_copy(rib[r], xi_hbm.at[pl.ds(0, CH)],
                                  sw[r]).wait()
            pltpu.make_async_copy(rjb[r], xj_hbm.at[pl.ds(0, CH)],
                                  sw[r]).wait()

    return gather


def _make_scatter(e_sl, bases):
    n_in = len(bases)
    epw = e_sl // NW
    nchunk = epw // CH
    nturn = ((nchunk + 3) // 4) * 4

    @functools.partial(
        pl.kernel,
        out_type=jax.ShapeDtypeStruct((NC, NPAD, DO), jnp.float32),
        mesh=_MESH,
        scratch_types=(
            [pltpu.VMEM((CH,), jnp.int32)] * 4
            + [pltpu.VMEM((CH, DO), jnp.float32)] * 4
            + [pltpu.VMEM_SHARED((NPAD, DO), jnp.float32)]
            + [pltpu.SemaphoreType.DMA] * 6
        ),
    )
    def scatter(*args):
        ms = args[:n_in]
        dst_hbm = args[n_in]
        zeros_hbm = args[n_in + 1]
        out_hbm = args[n_in + 2]
        (ix0, ix1, ix2, ix3, rw0, rw1, rw2, rw3, acc_sh,
         sl0, sl1, sl2, sl3, ss0, ss1) = args[n_in + 3:]
        ixb = (ix0, ix1, ix2, ix3)
        rwb = (rw0, rw1, rw2, rw3)
        sl = (sl0, sl1, sl2, sl3)
        ss = (ss0, ss1)

        cid = lax.axis_index("c")
        sid = lax.axis_index("s")
        wid = sid * NC + cid
        base = wid * epw

        pltpu.sync_copy(zeros_hbm.at[pl.ds(sid * NPS, NPS)],
                        acc_sh.at[pl.ds(sid * NPS, NPS)])
        plsc.subcore_barrier()

        for m_hbm, base0 in zip(ms, bases):
            def fire_load(c, q, m_hbm=m_hbm, base0=base0):
                off = base + c * CH
                pltpu.async_copy(dst_hbm.at[pl.ds(base0 + off, CH)],
                                 ixb[q], sl[q])
                pltpu.async_copy(m_hbm.at[pl.ds(off, CH)], rwb[q], sl[q])

            fire_load(0, 0)
            fire_load(1, 1)

            @pl.loop(0, nturn, step=4)
            def _turns(k, m_hbm=m_hbm, fire_load=fire_load):
                for db in range(4):
                    c = k + db
                    q = db
                    s = db % 2
                    qn = (db + 2) % 4

                    @pl.when(c < nchunk)
                    def _():
                        @pl.when(c >= 2)
                        def _():
                            pltpu.make_async_copy(
                                m_hbm.at[pl.ds(0, CH)], rwb[qn], ss[s]).wait()
                        pltpu.make_async_copy(
                            dst_hbm.at[pl.ds(0, CH)], ixb[q], sl[q]).wait()
                        pltpu.make_async_copy(
                            m_hbm.at[pl.ds(0, CH)], rwb[q], sl[q]).wait()
                        pltpu.async_copy(rwb[q], acc_sh.at[ixb[q]], ss[s],
                                         add=True)

                        @pl.when(c + 2 < nchunk)
                        def _():
                            fire_load(c + 2, qn)

            for s in range(2):
                pltpu.make_async_copy(m_hbm.at[pl.ds(0, CH)], rwb[s],
                                      ss[s]).wait()

        plsc.subcore_barrier()
        pltpu.sync_copy(acc_sh.at[pl.ds(sid * NPS, NPS)],
                        out_hbm.at[cid, pl.ds(sid * NPS, NPS)])

    return scatter


_gathers = [_make_gather(ESL, s * ESL) for s in range(SLICES)]
_scatter_a = _make_scatter(ESL, (0, ESL))
_scatter_b = _make_scatter(ESL, (2 * ESL, 3 * ESL))
_scatter_c = _make_scatter(ESL, (4 * ESL,))


def _mish(v):
    z = 1.0 + jnp.exp(jnp.minimum(v, 15.0))
    z2 = z * z
    return v * (z2 - 1.0) / (z2 + 1.0)


def _msg_body(xi_ref, xj_ref, ea_ref, w1a_ref, w1b_ref, b1_ref, w2_ref,
              b2_ref, wea_ref, web_ref, be1_ref, m_ref):
    xi = xi_ref[...].astype(jnp.bfloat16)
    xj = xj_ref[...].astype(jnp.bfloat16)
    dot = functools.partial(jnp.dot, preferred_element_type=jnp.float32)
    hp = dot(xi, w1a_ref[...]) + dot(xj, w1b_ref[...]) + b1_ref[...]
    h = dot(_mish(hp).astype(jnp.bfloat16), w2_ref[...]) + b2_ref[...]
    gp = (lax.dot_general(ea_ref[...].astype(jnp.bfloat16), wea_ref[...],
                          (((0,), (0,)), ((), ())),
                          preferred_element_type=jnp.float32)
          + dot(xj, web_ref[...]) + be1_ref[...])
    m_ref[...] = h * _mish(gp)


_BLK = 3200


def _tc_messages(s, xi, xj, ea_t, w1a, w1b, b1, w2, b2, wea, web, be1):
    e_sl = xi.shape[0]
    grid = (e_sl // _BLK,)
    blk0 = s * (e_sl // _BLK)
    def full(shape):
        return pl.BlockSpec(shape, lambda i: (0,) * len(shape))
    return pl.pallas_call(
        _msg_body,
        grid=grid,
        in_specs=[
            pl.BlockSpec((_BLK, DI), lambda i: (i, 0)),
            pl.BlockSpec((_BLK, DI), lambda i: (i, 0)),
            pl.BlockSpec((DE, _BLK), lambda i: (0, blk0 + i)),
            full((DI, DO)), full((DI, DO)), full((1, DO)),
            full((DO, DO)), full((1, DO)),
            full((DE, DO)), full((DI, DO)), full((1, DO)),
        ],
        out_specs=pl.BlockSpec((_BLK, DO), lambda i: (i, 0)),
        out_shape=jax.ShapeDtypeStruct((e_sl, DO), jnp.float32),
    )(xi, xj, ea_t, w1a, w1b, b1, w2, b2, wea, web, be1)


def _fin_body(pa_ref, pb_ref, pc_ref, x_ref, wr_ref, br_ref, g_ref, b_ref,
              out_ref):
    aggr = (pa_ref[0, :N, :] + pa_ref[1, :N, :]
            + pb_ref[0, :N, :] + pb_ref[1, :N, :]
            + pc_ref[0, :N, :] + pc_ref[1, :N, :])
    out = aggr + jnp.dot(x_ref[...], wr_ref[...],
                         preferred_element_type=jnp.float32) + br_ref[...]
    mu = jnp.mean(out, axis=0, keepdims=True)
    var = jnp.mean((out - mu) ** 2, axis=0, keepdims=True)
    out_ref[...] = (out - mu) * jax.lax.rsqrt(var + 1e-5) * g_ref[...] + b_ref[...]


def _tc_finish(pa, pb, pc, x, wr, br, gamma, beta):
    return pl.pallas_call(
        _fin_body,
        out_shape=jax.ShapeDtypeStruct((N, DO), jnp.float32),
    )(pa, pb, pc, x, wr, br, gamma, beta)


def kernel(x, edge_index, edge_attr, W1, b1, W2, b2, We1, be1, Wr, br,
           gamma, beta):
    srca = edge_index[0].astype(jnp.int32)
    dsta = edge_index[1].astype(jnp.int32)
    ea_t = edge_attr.T

    xpad = jnp.zeros((NPAD, DI), jnp.float32).at[:N].set(x)

    w1a = W1[:DI].astype(jnp.bfloat16)
    w1b = W1[DI:].astype(jnp.bfloat16)
    w2 = W2.astype(jnp.bfloat16)
    wea = We1[:DE].astype(jnp.bfloat16)
    web = We1[DE:].astype(jnp.bfloat16)
    b1r = b1.reshape(1, DO)
    b2r = b2.reshape(1, DO)
    be1r = be1.reshape(1, DO)

    ms = []
    for s in range(SLICES):
        xi_s, xj_s = _gathers[s](xpad, srca, dsta)
        ms.append(_tc_messages(s, xi_s, xj_s, ea_t,
                               w1a, w1b, b1r, w2, b2r, wea, web, be1r))

    zeros = jnp.zeros((NPAD, DO), jnp.float32)
    pa = _scatter_a(ms[0], ms[1], dsta, zeros)
    pb = _scatter_b(ms[2], ms[3], dsta, zeros)
    pc = _scatter_c(ms[4], dsta, zeros)

    out = _tc_finish(pa, pb, pc, x, Wr, br.reshape(1, DO),
                     gamma.reshape(1, DO), beta.reshape(1, DO))
    return (out, edge_index, edge_attr)
